# baseline (speedup 1.0000x reference)
import functools

import jax
import jax.numpy as jnp
from jax import lax
from jax.experimental import pallas as pl
from jax.experimental.pallas import tpu as pltpu
from jax.experimental.pallas import tpu_sc as plsc

N = 1024
NW = 32
SR = 2
NCH = 8
TILE = 128
NEG = float("-inf")

_GATHER_DNUMS = lax.GatherDimensionNumbers(
    offset_dims=(), collapsed_slice_dims=(0,), start_index_map=(0,))


def _lane_bcast(vec, j):
    idx = jnp.full((16, 1), j, dtype=jnp.int32)
    return lax.gather(vec, idx, dimension_numbers=_GATHER_DNUMS,
                      slice_sizes=(1,),
                      mode=lax.GatherScatterMode.PROMISE_IN_BOUNDS)


def _sc_make(s_rows, n_ot=8):
    srows = s_rows // NW
    nsr = srows // SR
    ntiles = N // TILE
    ng = n_ot * ntiles
    mesh = plsc.VectorSubcoreMesh(core_axis_name="c", subcore_axis_name="s")

    @functools.partial(
        pl.kernel,
        mesh=mesh,
        out_type=[
            jax.ShapeDtypeStruct((s_rows, n_ot * TILE), jnp.float32),
            jax.ShapeDtypeStruct((s_rows, n_ot * TILE), jnp.float32),
        ],
        scratch_types=[
            pltpu.VMEM((srows, N), jnp.float32),
            pltpu.VMEM((2, TILE, TILE), jnp.float32),
            pltpu.VMEM((2, TILE, TILE), jnp.float32),
            pltpu.VMEM((2, srows, TILE), jnp.float32),
            pltpu.VMEM((srows, TILE), jnp.float32),
            pltpu.SemaphoreType.DMA,
        ],
    )
    def k(t0_hbm, t1_hbm, out0_hbm, out1_hbm,
          a_v, b0_v, b1_v, o_acc, t1s_v, sem_b):
        wid = lax.axis_index("s") * 2 + lax.axis_index("c")
        s_off = pl.multiple_of(wid * srows, 8)

        def b_src(mat, g):
            ob = pl.multiple_of((g // ntiles) * TILE, TILE)
            mb = pl.multiple_of((g % ntiles) * TILE, TILE)
            return mat.at[pl.ds(mb, TILE), pl.ds(ob, TILE)]

        pltpu.async_copy(b_src(t0_hbm, 0), b0_v.at[0], sem_b)
        pltpu.async_copy(b_src(t1_hbm, 0), b1_v.at[0], sem_b)
        pltpu.sync_copy(t0_hbm.at[pl.ds(s_off, srows), :], a_v)

        def gstep(g, _):
            ob = g // ntiles
            mb = g % ntiles
            buf = g % 2
            pltpu.make_async_copy(b_src(t0_hbm, g), b0_v.at[buf], sem_b).wait()
            pltpu.make_async_copy(b_src(t1_hbm, g), b1_v.at[buf], sem_b).wait()
            gn = jnp.minimum(g + 1, ng - 1)
            pltpu.async_copy(b_src(t0_hbm, gn), b0_v.at[1 - buf], sem_b)
            pltpu.async_copy(b_src(t1_hbm, gn), b1_v.at[1 - buf], sem_b)

            fresh = mb == 0
            for oh in range(TILE // (NCH * 16)):
                oc0 = oh * NCH * 16

                def srstep(sr, _, oc0=oc0):
                    row0 = sr * SR

                    accs = tuple(
                        jnp.where(fresh, jnp.float32(NEG),
                                  o_acc[p, row0 + r, pl.ds(oc0 + c * 16, 16)])
                        for r in range(SR) for p in range(2)
                        for c in range(NCH))

                    def mcstep(mc, accs):
                        am0 = pl.multiple_of(mb * TILE + mc * 16, 16)
                        a16 = [a_v[row0 + r, pl.ds(am0, 16)]
                               for r in range(SR)]
                        accs = list(accs)
                        for j in range(16):
                            ml = mc * 16 + j
                            bb = [[b0_v[buf, ml, pl.ds(oc0 + c * 16, 16)]
                                   for c in range(NCH)],
                                  [b1_v[buf, ml, pl.ds(oc0 + c * 16, 16)]
                                   for c in range(NCH)]]
                            for r in range(SR):
                                av = _lane_bcast(a16[r], j)
                                for p in range(2):
                                    for c in range(NCH):
                                        i = (r * 2 + p) * NCH + c
                                        accs[i] = jnp.maximum(
                                            accs[i],
                                            jnp.minimum(av, bb[p][c]))
                        return tuple(accs)

                    accs = lax.fori_loop(0, TILE // 16, mcstep, accs)
                    for r in range(SR):
                        for p in range(2):
                            for c in range(NCH):
                                o_acc[p, row0 + r,
                                      pl.ds(oc0 + c * 16, 16)] = (
                                    accs[(r * 2 + p) * NCH + c])
                    return 0

                lax.fori_loop(0, nsr, srstep, 0)

            @pl.when(mb == ntiles - 1)
            def _():
                oc = pl.multiple_of(ob * TILE, TILE)
                pltpu.sync_copy(
                    t1_hbm.at[pl.ds(s_off, srows), pl.ds(oc, TILE)], t1s_v)

                def amstep(row, _):
                    for c in range(8):
                        t0v = a_v[row, pl.ds(oc + c * 16, 16)]
                        t1v = t1s_v[row, pl.ds(c * 16, 16)]
                        n0 = o_acc[0, row, pl.ds(c * 16, 16)]
                        n1 = o_acc[1, row, pl.ds(c * 16, 16)]
                        o_acc[0, row, pl.ds(c * 16, 16)] = t0v + n0 - t0v * n0
                        o_acc[1, row, pl.ds(c * 16, 16)] = t1v + n1 - t1v * n1
                    return 0

                lax.fori_loop(0, srows, amstep, 0)
                pltpu.sync_copy(
                    o_acc.at[0],
                    out0_hbm.at[pl.ds(s_off, srows), pl.ds(oc, TILE)])
                pltpu.sync_copy(
                    o_acc.at[1],
                    out1_hbm.at[pl.ds(s_off, srows), pl.ds(oc, TILE)])

            return 0

        lax.fori_loop(0, ng, gstep, 0)
        pltpu.make_async_copy(b_src(t0_hbm, ng - 1), b0_v.at[0], sem_b).wait()
        pltpu.make_async_copy(b_src(t1_hbm, ng - 1), b1_v.at[0], sem_b).wait()

    return k


SC_ROWS = 256
SC_OT = 7
TC_ROWS = N - SC_ROWS
S_B = 16

_sc_part = _sc_make(SC_ROWS, SC_OT)


def _tc_make(rows, row0, nt, ot0):
    def body(a_ref, b_ref, tp_ref, o_ref):
        acc_init = tuple(jnp.full((S_B, 128), NEG, dtype=jnp.float32)
                         for _ in range(2 * nt))

        def cstep(mc, accs):
            ac = a_ref[0, :, pl.ds(pl.multiple_of(mc * 128, 128), 128)]

            def bstep(mb, accs):
                accs = list(accs)
                base = pl.multiple_of(mc * 128 + mb * 8, 8)
                bc0 = b_ref[0, pl.ds(base, 8), :]
                bc1 = b_ref[1, pl.ds(base, 8), :]
                for j in range(8):
                    jj = mb * 8 + j
                    idx = jnp.full((S_B, 128), jj, dtype=jnp.int32)
                    a_bc = jnp.take_along_axis(ac, idx, axis=1)
                    for k in range(nt):
                        b0 = bc0[j:j + 1, k * 128:(k + 1) * 128]
                        b1 = bc1[j:j + 1, k * 128:(k + 1) * 128]
                        accs[k] = jnp.maximum(accs[k], jnp.minimum(a_bc, b0))
                        accs[nt + k] = jnp.maximum(
                            accs[nt + k], jnp.minimum(a_bc, b1))
                return tuple(accs)

            return jax.lax.fori_loop(0, 16, bstep, accs, unroll=16)

        accs = jax.lax.fori_loop(0, N // 128, cstep, acc_init)
        acc0 = jnp.concatenate(accs[:nt], axis=1)
        acc1 = jnp.concatenate(accs[nt:], axis=1)
        tp0 = tp_ref[0]
        tp1 = tp_ref[1]
        o_ref[0] = tp0 + acc0 - tp0 * acc0
        o_ref[1] = tp1 + acc1 - tp1 * acc1

    si0 = row0 // S_B
    w = nt * 128

    def call(t):
        return pl.pallas_call(
            body,
            grid=(rows // S_B,),
            in_specs=[
                pl.BlockSpec((1, S_B, N), lambda si: (0, si0 + si, 0)),
                pl.BlockSpec((2, N, w), lambda si: (0, 0, ot0)),
                pl.BlockSpec((2, S_B, w), lambda si: (0, si0 + si, ot0)),
            ],
            out_specs=pl.BlockSpec((2, S_B, w), lambda si: (0, si, 0)),
            out_shape=jax.ShapeDtypeStruct((2, rows, w), jnp.float32),
            compiler_params=pltpu.CompilerParams(
                dimension_semantics=("arbitrary",),
            ),
        )(t, t, t)

    return call


_tc_main = _tc_make(TC_ROWS, SC_ROWS, 8, 0)
_tc_strip = _tc_make(SC_ROWS, 0, 8 - SC_OT, SC_OT)


@jax.jit
def kernel(t):
    o0, o1 = _sc_part(t[0], t[1])
    strip = _tc_strip(t)
    main = _tc_main(t)
    top = jnp.concatenate([jnp.stack([o0, o1], axis=0), strip], axis=2)
    return jnp.concatenate([top, main], axis=1)


# speedup vs baseline: 1.1314x; 1.1314x over previous
import functools

import jax
import jax.numpy as jnp
from jax import lax
from jax.experimental import pallas as pl
from jax.experimental.pallas import tpu as pltpu
from jax.experimental.pallas import tpu_sc as plsc

N = 1024
NW = 32
SR = 2
NCH = 8
TILE = 128
NEG = float("-inf")

_GATHER_DNUMS = lax.GatherDimensionNumbers(
    offset_dims=(), collapsed_slice_dims=(0,), start_index_map=(0,))


def _lane_bcast(vec, j):
    idx = jnp.full((16, 1), j, dtype=jnp.int32)
    return lax.gather(vec, idx, dimension_numbers=_GATHER_DNUMS,
                      slice_sizes=(1,),
                      mode=lax.GatherScatterMode.PROMISE_IN_BOUNDS)


def _sc_make(s_rows):
    srows = s_rows // NW
    nsr = srows // SR
    ntiles = N // TILE
    ng = ntiles * ntiles
    mesh = plsc.VectorSubcoreMesh(core_axis_name="c", subcore_axis_name="s")

    @functools.partial(
        pl.kernel,
        mesh=mesh,
        out_type=[
            jax.ShapeDtypeStruct((s_rows, N), jnp.float32),
            jax.ShapeDtypeStruct((s_rows, N), jnp.float32),
        ],
        scratch_types=[
            pltpu.VMEM((srows, N), jnp.float32),
            pltpu.VMEM((2, TILE, TILE), jnp.float32),
            pltpu.VMEM((2, TILE, TILE), jnp.float32),
            pltpu.VMEM((2, srows, TILE), jnp.float32),
            pltpu.VMEM((srows, TILE), jnp.float32),
            pltpu.SemaphoreType.DMA,
        ],
    )
    def k(t0_hbm, t1_hbm, out0_hbm, out1_hbm,
          a_v, b0_v, b1_v, o_acc, t1s_v, sem_b):
        wid = lax.axis_index("s") * 2 + lax.axis_index("c")
        s_off = pl.multiple_of(wid * srows, 8)

        def b_src(mat, g):
            ob = pl.multiple_of((g // ntiles) * TILE, TILE)
            mb = pl.multiple_of((g % ntiles) * TILE, TILE)
            return mat.at[pl.ds(mb, TILE), pl.ds(ob, TILE)]

        pltpu.async_copy(b_src(t0_hbm, 0), b0_v.at[0], sem_b)
        pltpu.async_copy(b_src(t1_hbm, 0), b1_v.at[0], sem_b)
        pltpu.sync_copy(t0_hbm.at[pl.ds(s_off, srows), :], a_v)

        def gstep(g, _):
            ob = g // ntiles
            mb = g % ntiles
            buf = g % 2
            pltpu.make_async_copy(b_src(t0_hbm, g), b0_v.at[buf], sem_b).wait()
            pltpu.make_async_copy(b_src(t1_hbm, g), b1_v.at[buf], sem_b).wait()
            gn = jnp.minimum(g + 1, ng - 1)
            pltpu.async_copy(b_src(t0_hbm, gn), b0_v.at[1 - buf], sem_b)
            pltpu.async_copy(b_src(t1_hbm, gn), b1_v.at[1 - buf], sem_b)

            fresh = mb == 0
            for oh in range(TILE // (NCH * 16)):
                oc0 = oh * NCH * 16

                def srstep(sr, _, oc0=oc0):
                    row0 = sr * SR

                    accs = tuple(
                        jnp.where(fresh, jnp.float32(NEG),
                                  o_acc[p, row0 + r, pl.ds(oc0 + c * 16, 16)])
                        for r in range(SR) for p in range(2)
                        for c in range(NCH))

                    def mcstep(mc, accs):
                        am0 = pl.multiple_of(mb * TILE + mc * 16, 16)
                        a16 = [a_v[row0 + r, pl.ds(am0, 16)]
                               for r in range(SR)]
                        accs = list(accs)
                        for j in range(16):
                            ml = mc * 16 + j
                            bb = [[b0_v[buf, ml, pl.ds(oc0 + c * 16, 16)]
                                   for c in range(NCH)],
                                  [b1_v[buf, ml, pl.ds(oc0 + c * 16, 16)]
                                   for c in range(NCH)]]
                            for r in range(SR):
                                av = _lane_bcast(a16[r], j)
                                for p in range(2):
                                    for c in range(NCH):
                                        i = (r * 2 + p) * NCH + c
                                        accs[i] = jnp.maximum(
                                            accs[i],
                                            jnp.minimum(av, bb[p][c]))
                        return tuple(accs)

                    accs = lax.fori_loop(0, TILE // 16, mcstep, accs)
                    for r in range(SR):
                        for p in range(2):
                            for c in range(NCH):
                                o_acc[p, row0 + r,
                                      pl.ds(oc0 + c * 16, 16)] = (
                                    accs[(r * 2 + p) * NCH + c])
                    return 0

                lax.fori_loop(0, nsr, srstep, 0)

            @pl.when(mb == ntiles - 1)
            def _():
                oc = pl.multiple_of(ob * TILE, TILE)
                pltpu.sync_copy(
                    t1_hbm.at[pl.ds(s_off, srows), pl.ds(oc, TILE)], t1s_v)

                def amstep(row, _):
                    for c in range(8):
                        t0v = a_v[row, pl.ds(oc + c * 16, 16)]
                        t1v = t1s_v[row, pl.ds(c * 16, 16)]
                        n0 = o_acc[0, row, pl.ds(c * 16, 16)]
                        n1 = o_acc[1, row, pl.ds(c * 16, 16)]
                        o_acc[0, row, pl.ds(c * 16, 16)] = t0v + n0 - t0v * n0
                        o_acc[1, row, pl.ds(c * 16, 16)] = t1v + n1 - t1v * n1
                    return 0

                lax.fori_loop(0, srows, amstep, 0)
                pltpu.sync_copy(
                    o_acc.at[0],
                    out0_hbm.at[pl.ds(s_off, srows), pl.ds(oc, TILE)])
                pltpu.sync_copy(
                    o_acc.at[1],
                    out1_hbm.at[pl.ds(s_off, srows), pl.ds(oc, TILE)])

            return 0

        lax.fori_loop(0, ng, gstep, 0)
        pltpu.make_async_copy(b_src(t0_hbm, ng - 1), b0_v.at[0], sem_b).wait()
        pltpu.make_async_copy(b_src(t1_hbm, ng - 1), b1_v.at[0], sem_b).wait()

    return k


SC_ROWS = 256
TC_ROWS = N - SC_ROWS
S_B = 16

_sc_part = _sc_make(SC_ROWS)


NT = N // 128


def _tc_body(a_ref, b_ref, tp_ref, o_ref):
    acc_init = tuple(jnp.full((S_B, 128), NEG, dtype=jnp.float32)
                     for _ in range(2 * NT))

    def cstep(mc, accs):
        ac = a_ref[0, :, pl.ds(pl.multiple_of(mc * 128, 128), 128)]

        def bstep(mb, accs):
            accs = list(accs)
            base = pl.multiple_of(mc * 128 + mb * 8, 8)
            bc0 = b_ref[0, pl.ds(base, 8), :]
            bc1 = b_ref[1, pl.ds(base, 8), :]
            for j in range(8):
                jj = mb * 8 + j
                idx = jnp.full((S_B, 128), jj, dtype=jnp.int32)
                a_bc = jnp.take_along_axis(ac, idx, axis=1)
                for k in range(NT):
                    b0 = bc0[j:j + 1, k * 128:(k + 1) * 128]
                    b1 = bc1[j:j + 1, k * 128:(k + 1) * 128]
                    accs[k] = jnp.maximum(accs[k], jnp.minimum(a_bc, b0))
                    accs[NT + k] = jnp.maximum(
                        accs[NT + k], jnp.minimum(a_bc, b1))
            return tuple(accs)

        return jax.lax.fori_loop(0, 16, bstep, accs, unroll=16)

    accs = jax.lax.fori_loop(0, N // 128, cstep, acc_init)
    acc0 = jnp.concatenate(accs[:NT], axis=1)
    acc1 = jnp.concatenate(accs[NT:], axis=1)
    tp0 = tp_ref[0]
    tp1 = tp_ref[1]
    o_ref[0] = tp0 + acc0 - tp0 * acc0
    o_ref[1] = tp1 + acc1 - tp1 * acc1


_SI0 = SC_ROWS // S_B


def _tc_part(t):
    return pl.pallas_call(
        _tc_body,
        grid=(TC_ROWS // S_B,),
        in_specs=[
            pl.BlockSpec((1, S_B, N), lambda si: (0, _SI0 + si, 0)),
            pl.BlockSpec((2, N, N), lambda si: (0, 0, 0)),
            pl.BlockSpec((2, S_B, N), lambda si: (0, _SI0 + si, 0)),
        ],
        out_specs=pl.BlockSpec((2, S_B, N), lambda si: (0, si, 0)),
        out_shape=jax.ShapeDtypeStruct((2, TC_ROWS, N), jnp.float32),
        compiler_params=pltpu.CompilerParams(
            dimension_semantics=("arbitrary",),
        ),
    )(t, t, t)


@jax.jit
def kernel(t):
    o0, o1 = _sc_part(t[0], t[1])
    tc = _tc_part(t)
    sc = jnp.stack([o0, o1], axis=0)
    return jnp.concatenate([sc, tc], axis=1)
